# baseline (speedup 1.0000x reference)
import functools

import jax
import jax.numpy as jnp
from jax.experimental import pallas as pl
from jax.experimental.pallas import tpu as pltpu

_EPS = 1e-5
_CHUNK = 64


def _loss_body(p_ref, t_ref, out_ref, acc_ref, *, nsteps, n_total, num_image):
    i = pl.program_id(0)
    rows = p_ref.shape[1]

    def inner(k, carry):
        s_p, s_pt, s_t, s_f, s_ft = carry
        sl = pl.ds(k * _CHUNK, _CHUNK)
        p = p_ref[:, sl, :]
        tf = t_ref[:, sl, :].astype(jnp.float32)
        la = jnp.log(p + _EPS)
        lb = jnp.log((1.0 + _EPS) - p)
        ent = p * (la - lb) + lb
        s_p = s_p + jnp.sum(p, axis=0)
        s_pt = s_pt + jnp.sum(p * tf, axis=0)
        s_t = s_t + jnp.sum(tf, axis=0)
        s_f = s_f + jnp.sum(ent, axis=0)
        s_ft = s_ft + jnp.sum(ent * tf, axis=0)
        return s_p, s_pt, s_t, s_f, s_ft

    zero = jnp.zeros((_CHUNK, 256), jnp.float32)
    sums = jax.lax.fori_loop(0, rows // _CHUNK, inner,
                             (zero, zero, zero, zero, zero))
    block = jnp.stack(sums, axis=0)

    @pl.when(i == 0)
    def _init():
        acc_ref[...] = block

    @pl.when(i > 0)
    def _accum():
        acc_ref[...] = acc_ref[...] + block

    @pl.when(i == nsteps - 1)
    def _finish():
        acc = acc_ref[...]
        s_p = jnp.sum(acc[0])
        s_pt = jnp.sum(acc[1])
        s_t = jnp.sum(acc[2])
        s_f = jnp.sum(acc[3])
        s_ft = jnp.sum(acc[4])
        s_pneg = s_p - s_pt
        c_neg = n_total - s_t
        s_fneg = s_f - s_ft
        pos_avg = s_pt / jnp.maximum(s_t, 1.0)
        l1 = jnp.log(pos_avg + _EPS)
        l2 = jnp.log(1.0 - pos_avg + _EPS)
        entropy = -(s_fneg - l1 * s_pneg - l2 * (c_neg - s_pneg))
        out_ref[0, 0] = entropy / num_image


def kernel(pred, neg_target):
    n_total = pred.size
    num_image = pred.shape[0]
    block_imgs = 8
    nsteps = num_image // block_imgs
    out = pl.pallas_call(
        functools.partial(
            _loss_body,
            nsteps=nsteps,
            n_total=float(n_total),
            num_image=float(num_image),
        ),
        grid=(nsteps,),
        in_specs=[
            pl.BlockSpec((block_imgs, 256, 256), lambda i: (i, 0, 0)),
            pl.BlockSpec((block_imgs, 256, 256), lambda i: (i, 0, 0)),
        ],
        out_specs=pl.BlockSpec((1, 1), lambda i: (0, 0), memory_space=pltpu.SMEM),
        out_shape=jax.ShapeDtypeStruct((1, 1), jnp.float32),
        scratch_shapes=[pltpu.VMEM((5, _CHUNK, 256), jnp.float32)],
    )(pred, neg_target)
    return out[0, 0]


# speedup vs baseline: 1.0656x; 1.0656x over previous
import functools

import jax
import jax.numpy as jnp
from jax.experimental import pallas as pl
from jax.experimental.pallas import tpu as pltpu

_EPS = 1e-5
_CHUNK = 32


def _loss_body(p_ref, t_ref, out_ref, acc_ref, *, nsteps, n_total, num_image):
    i = pl.program_id(0)
    rows = p_ref.shape[1]

    def inner(k, carry):
        s_p, s_pt, s_t, s_f, s_ft = carry
        sl = pl.ds(k * _CHUNK, _CHUNK)
        p = p_ref[:, sl, :]
        tf = t_ref[:, sl, :].astype(jnp.float32)
        la = jnp.log(p + _EPS)
        lb = jnp.log((1.0 + _EPS) - p)
        ent = p * (la - lb) + lb
        s_p = s_p + jnp.sum(p, axis=0)
        s_pt = s_pt + jnp.sum(p * tf, axis=0)
        s_t = s_t + jnp.sum(tf, axis=0)
        s_f = s_f + jnp.sum(ent, axis=0)
        s_ft = s_ft + jnp.sum(ent * tf, axis=0)
        return s_p, s_pt, s_t, s_f, s_ft

    zero = jnp.zeros((_CHUNK, 256), jnp.float32)
    sums = jax.lax.fori_loop(0, rows // _CHUNK, inner,
                             (zero, zero, zero, zero, zero))
    block = jnp.stack(sums, axis=0)

    @pl.when(i == 0)
    def _init():
        acc_ref[...] = block

    @pl.when(i > 0)
    def _accum():
        acc_ref[...] = acc_ref[...] + block

    @pl.when(i == nsteps - 1)
    def _finish():
        acc = acc_ref[...]
        s_p = jnp.sum(acc[0])
        s_pt = jnp.sum(acc[1])
        s_t = jnp.sum(acc[2])
        s_f = jnp.sum(acc[3])
        s_ft = jnp.sum(acc[4])
        s_pneg = s_p - s_pt
        c_neg = n_total - s_t
        s_fneg = s_f - s_ft
        pos_avg = s_pt / jnp.maximum(s_t, 1.0)
        l1 = jnp.log(pos_avg + _EPS)
        l2 = jnp.log(1.0 - pos_avg + _EPS)
        entropy = -(s_fneg - l1 * s_pneg - l2 * (c_neg - s_pneg))
        out_ref[0, 0] = entropy / num_image


def kernel(pred, neg_target):
    n_total = pred.size
    num_image = pred.shape[0]
    block_imgs = 8
    nsteps = num_image // block_imgs
    out = pl.pallas_call(
        functools.partial(
            _loss_body,
            nsteps=nsteps,
            n_total=float(n_total),
            num_image=float(num_image),
        ),
        grid=(nsteps,),
        in_specs=[
            pl.BlockSpec((block_imgs, 256, 256), lambda i: (i, 0, 0)),
            pl.BlockSpec((block_imgs, 256, 256), lambda i: (i, 0, 0)),
        ],
        out_specs=pl.BlockSpec((1, 1), lambda i: (0, 0), memory_space=pltpu.SMEM),
        out_shape=jax.ShapeDtypeStruct((1, 1), jnp.float32),
        scratch_shapes=[pltpu.VMEM((5, _CHUNK, 256), jnp.float32)],
    )(pred, neg_target)
    return out[0, 0]
